# Initial kernel scaffold; baseline (speedup 1.0000x reference)
import jax
import jax.numpy as jnp
from jax.experimental import pallas as pl


def kernel(pred, target):
    raise NotImplementedError("write your pallas kernel here")


# speedup vs baseline: 2.2382x; 2.2382x over previous
import functools

import jax
import jax.numpy as jnp
from jax.experimental import pallas as pl
from jax.experimental.pallas import tpu as pltpu

_SMOOTHING = 0.1
_CONFIDENCE = 1.0 - _SMOOTHING


def _body(tgt_ref, x_ref, out_ref, m_ref, se_ref, ssum_ref, pt_ref,
          *, C, B, R, V):
    i = pl.program_id(0)
    j = pl.program_id(1)
    nv = pl.num_programs(1)

    x = x_ref[...]
    cols = jax.lax.broadcasted_iota(jnp.int32, (R, V), 1) + j * V
    valid = cols < C
    xm = jnp.where(valid, x, -jnp.inf)
    tgt = tgt_ref[...]
    pt_c = jnp.sum(jnp.where(cols == tgt, xm, 0.0), axis=1, keepdims=True)
    tile_max = jnp.max(xm, axis=1, keepdims=True)
    tile_sum = jnp.sum(jnp.where(valid, x, 0.0), axis=1, keepdims=True)

    @pl.when(j == 0)
    def _():
        m_ref[...] = tile_max
        se_ref[...] = jnp.sum(jnp.exp(xm - tile_max), axis=1, keepdims=True)
        ssum_ref[...] = tile_sum
        pt_ref[...] = pt_c

    @pl.when(j > 0)
    def _():
        m_old = m_ref[...]
        m_new = jnp.maximum(m_old, tile_max)
        se_ref[...] = (se_ref[...] * jnp.exp(m_old - m_new)
                       + jnp.sum(jnp.exp(xm - m_new), axis=1, keepdims=True))
        m_ref[...] = m_new
        ssum_ref[...] = ssum_ref[...] + tile_sum
        pt_ref[...] = pt_ref[...] + pt_c

    @pl.when(j == nv - 1)
    def _():
        lse = m_ref[...] + jnp.log(se_ref[...])
        s = _SMOOTHING / (C - 1)
        row_loss = -(s * (ssum_ref[...] - C * lse)
                     + (_CONFIDENCE - s) * (pt_ref[...] - lse))
        total = (jnp.sum(row_loss) * (1.0 / B)).reshape(1, 1)

        @pl.when(i == 0)
        def _():
            out_ref[...] = total

        @pl.when(i > 0)
        def _():
            out_ref[...] = out_ref[...] + total


def kernel(pred, target):
    B, C = pred.shape
    R = 256
    V = 2048
    nb = B // R
    nv = pl.cdiv(C, V)

    out = pl.pallas_call(
        functools.partial(_body, C=C, B=B, R=R, V=V),
        grid=(nb, nv),
        in_specs=[
            pl.BlockSpec((R, 1), lambda i, j: (i, 0)),
            pl.BlockSpec((R, V), lambda i, j: (i, j)),
        ],
        out_specs=pl.BlockSpec((1, 1), lambda i, j: (0, 0)),
        out_shape=jax.ShapeDtypeStruct((1, 1), jnp.float32),
        scratch_shapes=[
            pltpu.VMEM((R, 1), jnp.float32),
            pltpu.VMEM((R, 1), jnp.float32),
            pltpu.VMEM((R, 1), jnp.float32),
            pltpu.VMEM((R, 1), jnp.float32),
        ],
    )(target.reshape(B, 1).astype(jnp.int32), pred)
    return out[0, 0]
